# baseline (speedup 1.0000x reference)
import functools

import jax
import jax.numpy as jnp
from jax import lax
from jax.experimental import pallas as pl
from jax.experimental.pallas import tpu as pltpu
from jax.experimental.pallas import tpu_sc as plsc

D = 256
OMEGA = 0.04
PHI_MAX = 5.0
NC = 2
NS = 16
NW = NC * NS
L = 16
NJ = D // L

N_INJECT = 16384
N_QUERY = 4096
N_INJ_W = N_INJECT // NW
CHUNK = 64
NCHUNK = N_INJ_W // CHUNK
NBUF = 4
N_Q_W = N_QUERY // NW

_PI = 3.14159265358979323846
_TWO_PI = 2.0 * _PI
_SIN_C = (1.000000000e+00, -1.666666716e-01, 8.333333768e-03, -1.984127011e-04,
          2.755733249e-06, -2.505207242e-08, 1.605426347e-10, -7.583586537e-13,
          2.498001805e-15)
_COS_C = (1.000000000e+00, -5.000000000e-01, 4.166667163e-02, -1.388890552e-03,
          2.480187322e-05, -2.756005131e-07, 2.089865392e-09, -1.161400490e-11,
          5.262457137e-14, -2.220446049e-16)


def _sincos(x):
    xr = lax.rem(x, jnp.float32(_TWO_PI))
    xr = jnp.where(xr < 0, xr + jnp.float32(_TWO_PI), xr)
    y = xr - jnp.float32(_PI)
    u = y * y
    ps = jnp.float32(_SIN_C[-1])
    for c in _SIN_C[-2::-1]:
        ps = ps * u + jnp.float32(c)
    pc = jnp.float32(_COS_C[-1])
    for c in _COS_C[-2::-1]:
        pc = pc * u + jnp.float32(c)
    return -(y * ps), -pc


def _fused_body(b_hbm, rho_hbm, t_hbm, nidx_hbm, qidx_hbm,
                boosts_hbm, part_hbm, half_hbm,
                wr_v, wi_v, idx_v, qidx_v,
                rows0_v, rows1_v, rows2_v, rows3_v, qrows_v,
                p16_v, halves_v, res_v, acc_v, t16_v,
                sem0, sem1, sem2, sem3, qsem, ssem, bar_sem):
    c = lax.axis_index("c")
    s = lax.axis_index("s")
    wid = c * NS + s
    ibase = wid * N_INJ_W

    hidx = pltpu.async_copy(nidx_hbm.at[pl.ds(ibase, N_INJ_W)], idx_v, sem0)
    hq = pltpu.async_copy(qidx_hbm.at[pl.ds(wid * N_Q_W, N_Q_W)], qidx_v, qsem)
    hr = pltpu.async_copy(rho_hbm.at[pl.ds(ibase, N_INJ_W)], wr_v, ssem)
    ht = pltpu.async_copy(t_hbm.at[pl.ds(ibase, N_INJ_W)], wi_v, sem1)
    hidx.wait()

    bufs = (rows0_v, rows1_v, rows2_v, rows3_v)
    sems = (sem0, sem1, sem2, sem3)
    DEPTH = NBUF - 1
    ht.wait()
    hs = {}
    for k in range(DEPTH):
        hs[k] = pltpu.async_copy(
            b_hbm.at[idx_v.at[pl.ds(k * CHUNK, CHUNK)]], bufs[k], sems[k])
    hq.wait()
    qh = pltpu.async_copy(b_hbm.at[qidx_v], qrows_v, qsem)
    hr.wait()

    def wbody(i, carry):
        for e in range(2):
            sl = pl.ds(pl.multiple_of((i * 2 + e) * L, L), L)
            amp = wr_v[sl] * jnp.float32(1.0 / 16.0)
            sn, cs = _sincos(jnp.float32(OMEGA) * wi_v[sl])
            wr_v[sl] = amp * cs
            wi_v[sl] = amp * sn
        return carry
    lax.fori_loop(0, N_INJ_W // L // 2, wbody, 0)

    accs = tuple(jnp.zeros((L,), jnp.float32) for _ in range(2 * NJ))
    for k in range(NCHUNK):
        if k + DEPTH < NCHUNK:
            hs[k + DEPTH] = pltpu.async_copy(
                b_hbm.at[idx_v.at[pl.ds((k + DEPTH) * CHUNK, CHUNK)]],
                bufs[(k + DEPTH) % NBUF], sems[(k + DEPTH) % NBUF])
        hs.pop(k).wait()
        cur = bufs[k % NBUF]

        def rbody(r, acc_t, _k=k, _cur=cur):
            bvec = jnp.full((L,), _k * CHUNK + r, jnp.int32)
            wr = plsc.load_gather(wr_v, [bvec])
            wi = plsc.load_gather(wi_v, [bvec])
            rows = [_cur[r, pl.ds(j * L, L)] for j in range(NJ)]
            out = [acc_t[j] + wr * rows[j] for j in range(NJ)]
            out += [acc_t[NJ + j] + wi * rows[j] for j in range(NJ)]
            return tuple(out)
        accs = lax.fori_loop(0, CHUNK, rbody, accs)

    for j in range(2 * NJ):
        acc_v[pl.ds(j * L, L)] = accs[j]
    pltpu.sync_copy(acc_v, part_hbm.at[wid])

    plsc.subcore_barrier()

    NCOL = 128
    NSL = 2 * D // NCOL

    @pl.when(s < NSL)
    def _reduce_core_half():
        pltpu.sync_copy(
            part_hbm.at[pl.ds(c * NS, NS), pl.ds(s * NCOL, NCOL)], p16_v)

        def hbody(p, acc_t):
            return tuple(
                acc_t[j] + p16_v[p, pl.ds(j * L, L)] for j in range(NCOL // L))
        hacc = lax.fori_loop(
            0, NS, hbody, (jnp.zeros((L,), jnp.float32),) * (NCOL // L))
        for j in range(NCOL // L):
            acc_v[pl.ds(j * L, L)] = hacc[j]
        pltpu.sync_copy(acc_v.at[pl.ds(0, NCOL)],
                        half_hbm.at[c, pl.ds(s * NCOL, NCOL)])

    plsc.subcore_barrier()
    pltpu.semaphore_signal(bar_sem, 1, core_index=1 - c)
    pltpu.semaphore_wait(bar_sem, 1)
    qh.wait()

    pltpu.sync_copy(half_hbm, halves_v)
    raccs = tuple(
        halves_v[0, pl.ds(j * L, L)] + halves_v[1, pl.ds(j * L, L)]
        for j in range(2 * NJ))

    n2v = jnp.zeros((L,), jnp.float32)
    for j in range(2 * NJ):
        n2v = n2v + raccs[j] * raccs[j]
    n2 = jnp.full((L,), jnp.sum(n2v))
    i = plsc.bitcast(n2, jnp.int32)
    i = jnp.int32(0x5F3759DF) - lax.shift_right_arithmetic(i, 1)
    y = plsc.bitcast(i, jnp.float32)
    half = jnp.float32(0.5) * n2
    for _ in range(3):
        y = y * (jnp.float32(1.5) - half * y * y)
    scale = jnp.where(n2 > jnp.float32(PHI_MAX * PHI_MAX),
                      jnp.float32(PHI_MAX) * y, jnp.float32(1.0))
    rr = [raccs[j] * scale for j in range(NJ)]

    lanes = lax.iota(jnp.int32, L)

    def qgroup(g, carry):
        def qrow(h, carry2, _g=g):
            for e2 in range(2):
                e = h * 2 + e2
                r = _g * L + e
                acc = rr[0] * qrows_v[r, pl.ds(0, L)]
                for j in range(1, NJ):
                    acc = acc + rr[j] * qrows_v[r, pl.ds(j * L, L)]
                plsc.store_scatter(
                    t16_v, [lanes, jnp.full((L,), e, jnp.int32)], acc)
            return carry2
        lax.fori_loop(0, L // 2, qrow, 0)
        dots = t16_v[0, pl.ds(0, L)]
        for l in range(1, L):
            dots = dots + t16_v[l, pl.ds(0, L)]
        res_v[pl.ds(pl.multiple_of(g * L, L), L)] = dots
        return carry
    lax.fori_loop(0, N_Q_W // L, qgroup, 0)
    pltpu.sync_copy(res_v, boosts_hbm.at[pl.ds(wid * N_Q_W, N_Q_W)])


@functools.lru_cache(maxsize=None)
def _build():
    mesh = plsc.VectorSubcoreMesh(core_axis_name="c", subcore_axis_name="s")
    return pl.kernel(
        _fused_body,
        out_type=jax.ShapeDtypeStruct((N_QUERY,), jnp.float32),
        mesh=mesh,
        compiler_params=pltpu.CompilerParams(needs_layout_passes=False),
        scratch_types=[
            pltpu.HBM((NW, 2 * D), jnp.float32),
            pltpu.HBM((NC, 2 * D), jnp.float32),
            pltpu.VMEM((N_INJ_W,), jnp.float32),
            pltpu.VMEM((N_INJ_W,), jnp.float32),
            pltpu.VMEM((N_INJ_W,), jnp.int32),
            pltpu.VMEM((N_Q_W,), jnp.int32),
            pltpu.VMEM((CHUNK, D), jnp.float32),
            pltpu.VMEM((CHUNK, D), jnp.float32),
            pltpu.VMEM((CHUNK, D), jnp.float32),
            pltpu.VMEM((CHUNK, D), jnp.float32),
            pltpu.VMEM((N_Q_W, D), jnp.float32),
            pltpu.VMEM((NS, 128), jnp.float32),
            pltpu.VMEM((NC, 2 * D), jnp.float32),
            pltpu.VMEM((N_Q_W,), jnp.float32),
            pltpu.VMEM((2 * D,), jnp.float32),
            pltpu.VMEM((L, L), jnp.float32),
            pltpu.SemaphoreType.DMA,
            pltpu.SemaphoreType.DMA,
            pltpu.SemaphoreType.DMA,
            pltpu.SemaphoreType.DMA,
            pltpu.SemaphoreType.DMA,
            pltpu.SemaphoreType.DMA,
            pltpu.SemaphoreType.REGULAR,
        ],
    )


def kernel(B, rho, t, node_idx, query_idx):
    return _build()(B, rho, t, node_idx, query_idx)


# speedup vs baseline: 1.0327x; 1.0327x over previous
import functools

import jax
import jax.numpy as jnp
from jax import lax
from jax.experimental import pallas as pl
from jax.experimental.pallas import tpu as pltpu
from jax.experimental.pallas import tpu_sc as plsc

D = 256
OMEGA = 0.04
PHI_MAX = 5.0
NC = 2
NS = 16
NW = NC * NS
L = 16
NJ = D // L

N_INJECT = 16384
N_QUERY = 4096
N_INJ_W = N_INJECT // NW
CHUNK = 64
NCHUNK = N_INJ_W // CHUNK
NBUF = 4
N_Q_W = N_QUERY // NW

_PI = 3.14159265358979323846
_TWO_PI = 2.0 * _PI
_SIN_C = (1.000000000e+00, -1.666666716e-01, 8.333333768e-03, -1.984127011e-04,
          2.755733249e-06, -2.505207242e-08, 1.605426347e-10, -7.583586537e-13,
          2.498001805e-15)
_COS_C = (1.000000000e+00, -5.000000000e-01, 4.166667163e-02, -1.388890552e-03,
          2.480187322e-05, -2.756005131e-07, 2.089865392e-09, -1.161400490e-11,
          5.262457137e-14, -2.220446049e-16)


def _sincos(x):
    xr = lax.rem(x, jnp.float32(_TWO_PI))
    xr = jnp.where(xr < 0, xr + jnp.float32(_TWO_PI), xr)
    y = xr - jnp.float32(_PI)
    u = y * y
    ps = jnp.float32(_SIN_C[-1])
    for c in _SIN_C[-2::-1]:
        ps = ps * u + jnp.float32(c)
    pc = jnp.float32(_COS_C[-1])
    for c in _COS_C[-2::-1]:
        pc = pc * u + jnp.float32(c)
    return -(y * ps), -pc


def _fused_body(b_hbm, rho_hbm, t_hbm, nidx_hbm, qidx_hbm,
                boosts_hbm, part_hbm, half_hbm,
                wr_v, wi_v, idx_v, qidx_v,
                rows0_v, rows1_v, rows2_v, rows3_v, qrows_v,
                p16_v, halves_v, res_v, acc_v, t16_v,
                sem0, sem1, sem2, sem3, qsem, ssem, bar_sem):
    c = lax.axis_index("c")
    s = lax.axis_index("s")
    wid = c * NS + s
    ibase = wid * N_INJ_W

    hidx = pltpu.async_copy(nidx_hbm.at[pl.ds(ibase, N_INJ_W)], idx_v, sem0)
    hq = pltpu.async_copy(qidx_hbm.at[pl.ds(wid * N_Q_W, N_Q_W)], qidx_v, qsem)
    hr = pltpu.async_copy(rho_hbm.at[pl.ds(ibase, N_INJ_W)], wr_v, ssem)
    ht = pltpu.async_copy(t_hbm.at[pl.ds(ibase, N_INJ_W)], wi_v, sem1)
    hidx.wait()

    bufs = (rows0_v, rows1_v, rows2_v, rows3_v)
    sems = (sem0, sem1, sem2, sem3)
    DEPTH = NBUF - 1
    ht.wait()
    hs = {}
    for k in range(DEPTH):
        hs[k] = pltpu.async_copy(
            b_hbm.at[idx_v.at[pl.ds(k * CHUNK, CHUNK)]], bufs[k], sems[k])
    hq.wait()
    qh = pltpu.async_copy(b_hbm.at[qidx_v], qrows_v, qsem)
    hr.wait()

    def wbody(i, carry):
        for e in range(2):
            sl = pl.ds(pl.multiple_of((i * 2 + e) * L, L), L)
            amp = wr_v[sl] * jnp.float32(1.0 / 16.0)
            sn, cs = _sincos(jnp.float32(OMEGA) * wi_v[sl])
            wr_v[sl] = amp * cs
            wi_v[sl] = amp * sn
        return carry
    lax.fori_loop(0, N_INJ_W // L // 2, wbody, 0)

    accs = tuple(jnp.zeros((L,), jnp.float32) for _ in range(2 * NJ))
    for k in range(NCHUNK):
        if k + DEPTH < NCHUNK:
            hs[k + DEPTH] = pltpu.async_copy(
                b_hbm.at[idx_v.at[pl.ds((k + DEPTH) * CHUNK, CHUNK)]],
                bufs[(k + DEPTH) % NBUF], sems[(k + DEPTH) % NBUF])
        hs.pop(k).wait()
        cur = bufs[k % NBUF]

        def rbody(r, acc_t, _k=k, _cur=cur):
            bvec = jnp.full((L,), _k * CHUNK + r, jnp.int32)
            wr = plsc.load_gather(wr_v, [bvec])
            wi = plsc.load_gather(wi_v, [bvec])
            rows = [_cur[r, pl.ds(j * L, L)] for j in range(NJ)]
            out = [acc_t[j] + wr * rows[j] for j in range(NJ)]
            out += [acc_t[NJ + j] + wi * rows[j] for j in range(NJ)]
            return tuple(out)
        accs = lax.fori_loop(0, CHUNK, rbody, accs)

    for j in range(2 * NJ):
        acc_v[pl.ds(j * L, L)] = accs[j]
    pltpu.sync_copy(acc_v, part_hbm.at[wid])

    plsc.subcore_barrier()

    NCOL = 128
    NSL = 2 * D // NCOL

    @pl.when(s < NSL)
    def _reduce_core_half():
        pltpu.sync_copy(
            part_hbm.at[pl.ds(c * NS, NS), pl.ds(s * NCOL, NCOL)], p16_v)

        def hbody(p, acc_t):
            return tuple(
                acc_t[j] + p16_v[p, pl.ds(j * L, L)] for j in range(NCOL // L))
        hacc = lax.fori_loop(
            0, NS, hbody, (jnp.zeros((L,), jnp.float32),) * (NCOL // L))
        for j in range(NCOL // L):
            acc_v[pl.ds(j * L, L)] = hacc[j]
        pltpu.sync_copy(acc_v.at[pl.ds(0, NCOL)],
                        half_hbm.at[c, pl.ds(s * NCOL, NCOL)])

    plsc.subcore_barrier()
    pltpu.semaphore_signal(bar_sem, 1, core_index=1 - c)
    pltpu.semaphore_wait(bar_sem, 1)
    qh.wait()

    pltpu.sync_copy(half_hbm, halves_v)
    raccs = tuple(
        halves_v[0, pl.ds(j * L, L)] + halves_v[1, pl.ds(j * L, L)]
        for j in range(2 * NJ))

    n2v = jnp.zeros((L,), jnp.float32)
    for j in range(2 * NJ):
        n2v = n2v + raccs[j] * raccs[j]
    n2 = jnp.full((L,), jnp.sum(n2v))
    i = plsc.bitcast(n2, jnp.int32)
    i = jnp.int32(0x5F3759DF) - lax.shift_right_arithmetic(i, 1)
    y = plsc.bitcast(i, jnp.float32)
    half = jnp.float32(0.5) * n2
    for _ in range(3):
        y = y * (jnp.float32(1.5) - half * y * y)
    scale = jnp.where(n2 > jnp.float32(PHI_MAX * PHI_MAX),
                      jnp.float32(PHI_MAX) * y, jnp.float32(1.0))
    rr = [raccs[j] * scale for j in range(NJ)]

    lanes = lax.iota(jnp.int32, L)

    def qgroup(g, carry):
        def qrow(h, carry2, _g=g):
            for e2 in range(2):
                e = h * 2 + e2
                r = _g * L + e
                terms = [rr[j] * qrows_v[r, pl.ds(j * L, L)] for j in range(NJ)]
                while len(terms) > 1:
                    terms = [terms[i] + terms[i + 1]
                             for i in range(0, len(terms), 2)]
                plsc.store_scatter(
                    t16_v, [lanes, jnp.full((L,), e, jnp.int32)], terms[0])
            return carry2
        lax.fori_loop(0, L // 2, qrow, 0)
        cols = [t16_v[l, pl.ds(0, L)] for l in range(L)]
        while len(cols) > 1:
            cols = [cols[i] + cols[i + 1] for i in range(0, len(cols), 2)]
        res_v[pl.ds(pl.multiple_of(g * L, L), L)] = cols[0]
        return carry
    lax.fori_loop(0, N_Q_W // L, qgroup, 0)
    pltpu.sync_copy(res_v, boosts_hbm.at[pl.ds(wid * N_Q_W, N_Q_W)])


@functools.lru_cache(maxsize=None)
def _build():
    mesh = plsc.VectorSubcoreMesh(core_axis_name="c", subcore_axis_name="s")
    return pl.kernel(
        _fused_body,
        out_type=jax.ShapeDtypeStruct((N_QUERY,), jnp.float32),
        mesh=mesh,
        compiler_params=pltpu.CompilerParams(needs_layout_passes=False),
        scratch_types=[
            pltpu.HBM((NW, 2 * D), jnp.float32),
            pltpu.HBM((NC, 2 * D), jnp.float32),
            pltpu.VMEM((N_INJ_W,), jnp.float32),
            pltpu.VMEM((N_INJ_W,), jnp.float32),
            pltpu.VMEM((N_INJ_W,), jnp.int32),
            pltpu.VMEM((N_Q_W,), jnp.int32),
            pltpu.VMEM((CHUNK, D), jnp.float32),
            pltpu.VMEM((CHUNK, D), jnp.float32),
            pltpu.VMEM((CHUNK, D), jnp.float32),
            pltpu.VMEM((CHUNK, D), jnp.float32),
            pltpu.VMEM((N_Q_W, D), jnp.float32),
            pltpu.VMEM((NS, 128), jnp.float32),
            pltpu.VMEM((NC, 2 * D), jnp.float32),
            pltpu.VMEM((N_Q_W,), jnp.float32),
            pltpu.VMEM((2 * D,), jnp.float32),
            pltpu.VMEM((L, L), jnp.float32),
            pltpu.SemaphoreType.DMA,
            pltpu.SemaphoreType.DMA,
            pltpu.SemaphoreType.DMA,
            pltpu.SemaphoreType.DMA,
            pltpu.SemaphoreType.DMA,
            pltpu.SemaphoreType.DMA,
            pltpu.SemaphoreType.REGULAR,
        ],
    )


def kernel(B, rho, t, node_idx, query_idx):
    return _build()(B, rho, t, node_idx, query_idx)
